# Initial kernel scaffold; baseline (speedup 1.0000x reference)
import jax
import jax.numpy as jnp
from jax.experimental import pallas as pl


def kernel(x, g):
    raise NotImplementedError("write your pallas kernel here")


# speedup vs baseline: 1.5768x; 1.5768x over previous
import jax
import jax.numpy as jnp
from jax import lax
from jax.experimental import pallas as pl
from jax.experimental.pallas import tpu as pltpu, tpu_sc as plsc

N, C, H, W = 4, 32, 384, 384
HW = H * W
NPIX = N * HW
NC, NS, L = 2, 16, 16
NW = NC * NS
PPW = HW // NW
BP = 512
NBLK = PPW // BP
TBLK = N * NBLK
NSUB = BP // 128


def _body(xt, gx, gy, out, gxv, gyv, i00, i10, i01, i11, w00, w10, w01, w11,
          v00, v10, v01, v11, ob, gsem, osem):
    cid = lax.axis_index("c")
    sid = lax.axis_index("s")
    wid = sid * NC + cid
    q0w = wid * PPW
    iota = lax.iota(jnp.int32, L)

    def block(t, carry):
        n = t // NBLK
        b = t - n * NBLK
        q0 = q0w + b * BP
        p0 = pl.multiple_of(n * HW + q0, BP)
        pltpu.sync_copy(gx.at[pl.ds(p0, BP)], gxv)
        pltpu.sync_copy(gy.at[pl.ds(p0, BP)], gyv)
        rowbase = n * HW
        for q in range(BP // L):
            sl = pl.ds(q * L, L)
            ix = (gxv[sl] + 1.0) * (0.5 * (W - 1))
            iy = (gyv[sl] + 1.0) * (0.5 * (H - 1))
            tx = ix.astype(jnp.int32)
            ty = iy.astype(jnp.int32)
            ix0 = tx - jnp.where(ix < tx.astype(jnp.float32), 1, 0)
            iy0 = ty - jnp.where(iy < ty.astype(jnp.float32), 1, 0)
            wx1 = ix - ix0.astype(jnp.float32)
            wy1 = iy - iy0.astype(jnp.float32)
            wx0 = 1.0 - wx1
            wy0 = 1.0 - wy1
            vx0 = jnp.where((ix0 >= 0) & (ix0 <= W - 1), 1.0, 0.0)
            vx1 = jnp.where((ix0 >= -1) & (ix0 <= W - 2), 1.0, 0.0)
            vy0 = jnp.where((iy0 >= 0) & (iy0 <= H - 1), 1.0, 0.0)
            vy1 = jnp.where((iy0 >= -1) & (iy0 <= H - 2), 1.0, 0.0)
            cx0 = jnp.clip(ix0, 0, W - 1)
            cx1 = jnp.clip(ix0 + 1, 0, W - 1)
            r0 = rowbase + jnp.clip(iy0, 0, H - 1) * W
            r1 = rowbase + jnp.clip(iy0 + 1, 0, H - 1) * W
            r, cc = q // 8, (q % 8) * L
            csl = pl.ds(cc, L)
            i00[r, csl] = r0 + cx0
            i10[r, csl] = r0 + cx1
            i01[r, csl] = r1 + cx0
            i11[r, csl] = r1 + cx1
            w00[sl] = wx0 * wy0 * vx0 * vy0
            w10[sl] = wx1 * wy0 * vx1 * vy0
            w01[sl] = wx0 * wy1 * vx0 * vy1
            w11[sl] = wx1 * wy1 * vx1 * vy1
        descs = []
        for i in range(NSUB):
            dsl = pl.ds(i * 128, 128)
            for iref, vref in ((i00, v00), (i10, v10), (i01, v01), (i11, v11)):
                descs.append(pltpu.async_copy(xt.at[iref.at[i]], vref.at[dsl], gsem))
        for d in descs:
            d.wait()

        def grp(j, carry2):
            rows = j * L + iota
            jsl = pl.ds(j * L, L)
            w00v = w00[jsl]
            w10v = w10[jsl]
            w01v = w01[jsl]
            w11v = w11[jsl]
            for c in range(C):
                colc = jnp.full((L,), c, jnp.int32)
                a = plsc.load_gather(v00, [rows, colc]) * w00v
                a = a + plsc.load_gather(v10, [rows, colc]) * w10v
                a = a + plsc.load_gather(v01, [rows, colc]) * w01v
                a = a + plsc.load_gather(v11, [rows, colc]) * w11v
                ob[pl.ds(c * BP + j * L, L)] = a
            return carry2

        lax.fori_loop(0, BP // L, grp, 0)
        odescs = []
        for c in range(C):
            off = pl.multiple_of((n * C + c) * HW + q0, BP)
            odescs.append(pltpu.async_copy(ob.at[pl.ds(c * BP, BP)],
                                           out.at[pl.ds(off, BP)], osem))
        for d in odescs:
            d.wait()
        return carry

    lax.fori_loop(0, TBLK, block, 0)


def kernel(x, g):
    xt = jnp.transpose(x, (0, 2, 3, 1)).reshape(NPIX, C)
    gx = jnp.reshape(g[..., 0], (NPIX,))
    gy = jnp.reshape(g[..., 1], (NPIX,))
    mesh = plsc.VectorSubcoreMesh(core_axis_name="c", subcore_axis_name="s")
    f = pl.kernel(
        _body,
        out_type=jax.ShapeDtypeStruct((N * C * HW,), jnp.float32),
        mesh=mesh,
        compiler_params=pltpu.CompilerParams(
            needs_layout_passes=False, use_tc_tiling_on_sc=False),
        scratch_types=[
            pltpu.VMEM((BP,), jnp.float32),
            pltpu.VMEM((BP,), jnp.float32),
            pltpu.VMEM((NSUB, 128), jnp.int32),
            pltpu.VMEM((NSUB, 128), jnp.int32),
            pltpu.VMEM((NSUB, 128), jnp.int32),
            pltpu.VMEM((NSUB, 128), jnp.int32),
            pltpu.VMEM((BP,), jnp.float32),
            pltpu.VMEM((BP,), jnp.float32),
            pltpu.VMEM((BP,), jnp.float32),
            pltpu.VMEM((BP,), jnp.float32),
            pltpu.VMEM((BP, C), jnp.float32),
            pltpu.VMEM((BP, C), jnp.float32),
            pltpu.VMEM((BP, C), jnp.float32),
            pltpu.VMEM((BP, C), jnp.float32),
            pltpu.VMEM((C * BP,), jnp.float32),
            pltpu.SemaphoreType.DMA,
            pltpu.SemaphoreType.DMA,
        ],
    )
    y = f(xt, gx, gy)
    return y.reshape(N, C, H, W)


# speedup vs baseline: 4.5387x; 2.8785x over previous
import jax
import jax.numpy as jnp
from jax import lax
from jax.experimental import pallas as pl
from jax.experimental.pallas import tpu as pltpu, tpu_sc as plsc

N, C, H, W = 4, 32, 384, 384
HW = H * W
NPIX = N * HW
NC, NS, L = 2, 16, 16
NW = NC * NS
PPW = HW // NW
BP = 256
NBLK = PPW // BP
TBLK = N * NBLK
NSUB = BP // 128
NQ = BP // L


def _body(xt, gx, gy, out, gxv, gyv, i00, i10, i01, i11, w00, w10, w01, w11,
          v00, v10, v01, v11, ob, gsem0, gsem1, osem0, osem1):
    cid = lax.axis_index("c")
    sid = lax.axis_index("s")
    wid = sid * NC + cid
    iota = lax.iota(jnp.int32, L)
    gsems = (gsem0, gsem1)
    osems = (osem0, osem1)

    def fire(bp, tn):
        bat = tn // NBLK
        r = tn - bat * NBLK

        @pl.when(r == 0)
        def _():
            goff = pl.multiple_of(bat * HW + wid * PPW, PPW)
            pltpu.sync_copy(gx.at[pl.ds(goff, PPW)], gxv)
            pltpu.sync_copy(gy.at[pl.ds(goff, PPW)], gyv)

        rowbase = bat * HW

        def qstep(q, carry):
            sl = pl.ds(r * BP + q * L, L)
            ix = (gxv[sl] + 1.0) * (0.5 * (W - 1))
            iy = (gyv[sl] + 1.0) * (0.5 * (H - 1))
            tx = ix.astype(jnp.int32)
            ty = iy.astype(jnp.int32)
            ix0 = tx - jnp.where(ix < tx.astype(jnp.float32), 1, 0)
            iy0 = ty - jnp.where(iy < ty.astype(jnp.float32), 1, 0)
            wx1 = ix - ix0.astype(jnp.float32)
            wy1 = iy - iy0.astype(jnp.float32)
            wx0 = 1.0 - wx1
            wy0 = 1.0 - wy1
            vx0 = jnp.where((ix0 >= 0) & (ix0 <= W - 1), 1.0, 0.0)
            vx1 = jnp.where((ix0 >= -1) & (ix0 <= W - 2), 1.0, 0.0)
            vy0 = jnp.where((iy0 >= 0) & (iy0 <= H - 1), 1.0, 0.0)
            vy1 = jnp.where((iy0 >= -1) & (iy0 <= H - 2), 1.0, 0.0)
            cx0 = jnp.clip(ix0, 0, W - 1)
            cx1 = jnp.clip(ix0 + 1, 0, W - 1)
            r0 = rowbase + jnp.clip(iy0, 0, H - 1) * W
            r1 = rowbase + jnp.clip(iy0 + 1, 0, H - 1) * W
            sub = q // (128 // L)
            cc = (q - sub * (128 // L)) * L
            csl = pl.ds(cc, L)
            i00[bp, sub, csl] = r0 + cx0
            i10[bp, sub, csl] = r0 + cx1
            i01[bp, sub, csl] = r1 + cx0
            i11[bp, sub, csl] = r1 + cx1
            qsl = pl.ds(q * L, L)
            w00[bp, qsl] = wx0 * wy0 * vx0 * vy0
            w10[bp, qsl] = wx1 * wy0 * vx1 * vy0
            w01[bp, qsl] = wx0 * wy1 * vx0 * vy1
            w11[bp, qsl] = wx1 * wy1 * vx1 * vy1
            return carry

        lax.fori_loop(0, NQ, qstep, 0)
        for i in range(NSUB):
            dsl = pl.ds(i * 128, 128)
            for iref, vref in ((i00, v00), (i10, v10), (i01, v01), (i11, v11)):
                pltpu.async_copy(xt.at[iref.at[bp, i]], vref.at[bp, dsl],
                                 gsems[bp])

    def drain_gathers(bp):
        for vref in (v00, v10, v01, v11):
            pltpu.make_async_copy(xt.at[pl.ds(0, BP)], vref.at[bp],
                                  gsems[bp]).wait()

    def combine(bp, t):
        parv = jnp.full((L,), bp, jnp.int32)

        def jstep(j, carry):
            rows = j * L + iota
            jsl = pl.ds(j * L, L)
            w00v = w00[bp, jsl]
            w10v = w10[bp, jsl]
            w01v = w01[bp, jsl]
            w11v = w11[bp, jsl]
            for c0 in range(C):
                colv = (c0 + iota) & (C - 1)
                a = plsc.load_gather(v00, [parv, rows, colv]) * w00v
                a = a + plsc.load_gather(v10, [parv, rows, colv]) * w10v
                a = a + plsc.load_gather(v01, [parv, rows, colv]) * w01v
                a = a + plsc.load_gather(v11, [parv, rows, colv]) * w11v
                plsc.store_scatter(ob, [parv, colv * BP + rows], a)
            return carry

        lax.fori_loop(0, NQ, jstep, 0)

    def fire_outs(bp, t):
        bat = t // NBLK
        q0 = (t - bat * NBLK) * BP + wid * PPW
        for c in range(C):
            off = pl.multiple_of((bat * C + c) * HW + q0, BP)
            pltpu.async_copy(ob.at[bp, pl.ds(c * BP, BP)],
                             out.at[pl.ds(off, BP)], osems[bp])

    def drain_outs(bp):
        pltpu.make_async_copy(out.at[pl.ds(0, C * BP)], ob.at[bp],
                              osems[bp]).wait()

    fire(0, 0)

    def block(t, carry):
        for par in range(2):
            @pl.when(lax.rem(t, 2) == par)
            def _():
                @pl.when(t >= 2)
                def _():
                    drain_outs(par)

                @pl.when(t + 1 < TBLK)
                def _():
                    fire(1 - par, t + 1)

                drain_gathers(par)
                combine(par, t)
                fire_outs(par, t)
        return carry

    lax.fori_loop(0, TBLK, block, 0)
    drain_outs(TBLK % 2)
    drain_outs(1 - TBLK % 2)


def kernel(x, g):
    xt = jnp.transpose(x, (0, 2, 3, 1)).reshape(NPIX, C)
    gx = jnp.reshape(g[..., 0], (NPIX,))
    gy = jnp.reshape(g[..., 1], (NPIX,))
    mesh = plsc.VectorSubcoreMesh(core_axis_name="c", subcore_axis_name="s")
    f = pl.kernel(
        _body,
        out_type=jax.ShapeDtypeStruct((N * C * HW,), jnp.float32),
        mesh=mesh,
        compiler_params=pltpu.CompilerParams(
            needs_layout_passes=False, use_tc_tiling_on_sc=False),
        scratch_types=[
            pltpu.VMEM((PPW,), jnp.float32),
            pltpu.VMEM((PPW,), jnp.float32),
            pltpu.VMEM((2, NSUB, 128), jnp.int32),
            pltpu.VMEM((2, NSUB, 128), jnp.int32),
            pltpu.VMEM((2, NSUB, 128), jnp.int32),
            pltpu.VMEM((2, NSUB, 128), jnp.int32),
            pltpu.VMEM((2, BP), jnp.float32),
            pltpu.VMEM((2, BP), jnp.float32),
            pltpu.VMEM((2, BP), jnp.float32),
            pltpu.VMEM((2, BP), jnp.float32),
            pltpu.VMEM((2, BP, C), jnp.float32),
            pltpu.VMEM((2, BP, C), jnp.float32),
            pltpu.VMEM((2, BP, C), jnp.float32),
            pltpu.VMEM((2, BP, C), jnp.float32),
            pltpu.VMEM((2, C * BP), jnp.float32),
            pltpu.SemaphoreType.DMA,
            pltpu.SemaphoreType.DMA,
            pltpu.SemaphoreType.DMA,
            pltpu.SemaphoreType.DMA,
        ],
    )
    y = f(xt, gx, gy)
    return y.reshape(N, C, H, W)


# speedup vs baseline: 5.0286x; 1.1079x over previous
import jax
import jax.numpy as jnp
from jax import lax
from jax.experimental import pallas as pl
from jax.experimental.pallas import tpu as pltpu, tpu_sc as plsc

N, C, H, W = 4, 32, 384, 384
HW = H * W
NPIX = N * HW
NC, NS, L = 2, 16, 16
NW = NC * NS
PPW = HW // NW
BP = 256
NBLK = PPW // BP
TBLK = N * NBLK
NSUB = BP // 128
NQ = BP // L


def _body(xt, gx, gy, out, gxv, gyv, i00, i10, i01, i11, w00, w10, w01, w11,
          v00, v10, v01, v11, ob, gsem0, gsem1, osem0, osem1):
    cid = lax.axis_index("c")
    sid = lax.axis_index("s")
    wid = sid * NC + cid
    iota = lax.iota(jnp.int32, L)
    gsems = (gsem0, gsem1)
    osems = (osem0, osem1)

    def fire(bp, tn):
        bat = tn // NBLK
        r = tn - bat * NBLK

        @pl.when(r == 0)
        def _():
            goff = pl.multiple_of(bat * HW + wid * PPW, PPW)
            pltpu.sync_copy(gx.at[pl.ds(goff, PPW)], gxv)
            pltpu.sync_copy(gy.at[pl.ds(goff, PPW)], gyv)

        rowbase = bat * HW

        def qstep(q, carry):
            sl = pl.ds(r * BP + q * L, L)
            ix = (gxv[sl] + 1.0) * (0.5 * (W - 1))
            iy = (gyv[sl] + 1.0) * (0.5 * (H - 1))
            tx = ix.astype(jnp.int32)
            ty = iy.astype(jnp.int32)
            ix0 = tx - jnp.where(ix < tx.astype(jnp.float32), 1, 0)
            iy0 = ty - jnp.where(iy < ty.astype(jnp.float32), 1, 0)
            wx1 = ix - ix0.astype(jnp.float32)
            wy1 = iy - iy0.astype(jnp.float32)
            wx0 = 1.0 - wx1
            wy0 = 1.0 - wy1
            vx0 = jnp.where((ix0 >= 0) & (ix0 <= W - 1), 1.0, 0.0)
            vx1 = jnp.where((ix0 >= -1) & (ix0 <= W - 2), 1.0, 0.0)
            vy0 = jnp.where((iy0 >= 0) & (iy0 <= H - 1), 1.0, 0.0)
            vy1 = jnp.where((iy0 >= -1) & (iy0 <= H - 2), 1.0, 0.0)
            cx0 = jnp.clip(ix0, 0, W - 1)
            cx1 = jnp.clip(ix0 + 1, 0, W - 1)
            r0 = rowbase + jnp.clip(iy0, 0, H - 1) * W
            r1 = rowbase + jnp.clip(iy0 + 1, 0, H - 1) * W
            sub = q // (128 // L)
            cc = (q - sub * (128 // L)) * L
            csl = pl.ds(cc, L)
            i00[bp, sub, csl] = r0 + cx0
            i10[bp, sub, csl] = r0 + cx1
            i01[bp, sub, csl] = r1 + cx0
            i11[bp, sub, csl] = r1 + cx1
            qsl = pl.ds(q * L, L)
            w00[bp, qsl] = wx0 * wy0 * vx0 * vy0
            w10[bp, qsl] = wx1 * wy0 * vx1 * vy0
            w01[bp, qsl] = wx0 * wy1 * vx0 * vy1
            w11[bp, qsl] = wx1 * wy1 * vx1 * vy1
            return carry

        lax.fori_loop(0, NQ, qstep, 0)
        for i in range(NSUB):
            dsl = pl.ds(i * 128, 128)
            for iref, vref in ((i00, v00), (i10, v10), (i01, v01), (i11, v11)):
                pltpu.async_copy(xt.at[iref.at[bp, i]], vref.at[bp, dsl],
                                 gsems[bp])

    def drain_gathers(bp):
        for vref in (v00, v10, v01, v11):
            pltpu.make_async_copy(xt.at[pl.ds(0, BP)], vref.at[bp],
                                  gsems[bp]).wait()

    def combine(bp, t):
        parv = jnp.full((L,), bp, jnp.int32)

        def jstep(j, carry):
            rows = j * L + iota
            jsl = pl.ds(j * L, L)
            w00v = w00[bp, jsl]
            w10v = w10[bp, jsl]
            w01v = w01[bp, jsl]
            w11v = w11[bp, jsl]
            for c0 in range(C):
                colv = (c0 + iota) & (C - 1)
                a = plsc.load_gather(v00, [parv, rows, colv]) * w00v
                a = a + plsc.load_gather(v10, [parv, rows, colv]) * w10v
                a = a + plsc.load_gather(v01, [parv, rows, colv]) * w01v
                a = a + plsc.load_gather(v11, [parv, rows, colv]) * w11v
                plsc.store_scatter(ob, [parv, colv * BP + rows], a)
            return carry

        lax.fori_loop(0, NQ, jstep, 0)

    def fire_outs(bp, t):
        bat = t // NBLK
        q0 = (t - bat * NBLK) * BP + wid * PPW
        for c in range(C):
            off = pl.multiple_of((bat * C + c) * HW + q0, BP)
            pltpu.async_copy(ob.at[bp, pl.ds(c * BP, BP)],
                             out.at[pl.ds(off, BP)], osems[bp])

    def drain_outs(bp):
        pltpu.make_async_copy(out.at[pl.ds(0, C * BP)], ob.at[bp],
                              osems[bp]).wait()

    fire(0, 0)

    def block(t, carry):
        for par in range(2):
            @pl.when(lax.rem(t, 2) == par)
            def _():
                @pl.when(t >= 2)
                def _():
                    drain_outs(par)

                @pl.when(t + 1 < TBLK)
                def _():
                    fire(1 - par, t + 1)

                drain_gathers(par)
                combine(par, t)
                fire_outs(par, t)
        return carry

    lax.fori_loop(0, TBLK, block, 0)
    drain_outs(TBLK % 2)
    drain_outs(1 - TBLK % 2)


TB = 18
TILE_ELEMS = C * 8 * 128


def _tbody(x, out, ibuf, obuf, rsem0, rsem1, wsem):
    cid = lax.axis_index("c")
    sid = lax.axis_index("s")
    wid = sid * NC + cid
    iota = lax.iota(jnp.int32, L)
    rsems = (rsem0, rsem1)
    WT = W // 128
    HS = H // 8
    PERN = HS * WT

    def nhw(blk):
        n = blk // PERN
        rem = blk - n * PERN
        hs = rem // WT
        wt = rem - hs * WT
        return n, hs * 8, wt * 128

    def fire_reads(bp, b):
        n, h0, w0 = nhw(wid * TB + b)
        for c in range(C):
            pltpu.async_copy(x.at[n, c, pl.ds(h0, 8), pl.ds(w0, 128)],
                             ibuf.at[bp, c], rsems[bp])

    def drain_reads(bp):
        pltpu.make_async_copy(x.at[0, pl.ds(0, C), pl.ds(0, 8), pl.ds(0, 128)],
                              ibuf.at[bp], rsems[bp]).wait()

    fire_reads(0, 0)

    def block(b, carry):
        for par in range(2):
            @pl.when(lax.rem(b, 2) == par)
            def _():
                @pl.when(b + 1 < TB)
                def _():
                    fire_reads(1 - par, b + 1)

                drain_reads(par)

                @pl.when(b >= 1)
                def _():
                    pltpu.make_async_copy(out.at[pl.ds(0, C * 8 * 128)],
                                          obuf, wsem).wait()

                parv = jnp.full((L,), par, jnp.int32)

                def hq(k, carry2):
                    h = k // 8
                    q = k - h * 8
                    wv = q * L + iota
                    hv = jnp.full((L,), h, jnp.int32)
                    obase = h * 4096 + wv * C
                    for c0 in range(C):
                        cv = (c0 + iota) & (C - 1)
                        a = plsc.load_gather(ibuf, [parv, cv, hv, wv])
                        plsc.store_scatter(obuf, [obase + cv], a)
                    return carry2

                lax.fori_loop(0, 64, hq, 0)
                n, h0, w0 = nhw(wid * TB + b)
                for h in range(8):
                    off = pl.multiple_of(
                        (n * HW + (h0 + h) * W + w0) * C, 4096)
                    pltpu.async_copy(obuf.at[pl.ds(h * 4096, 4096)],
                                     out.at[pl.ds(off, 4096)], wsem)
        return carry

    lax.fori_loop(0, TB, block, 0)
    pltpu.make_async_copy(out.at[pl.ds(0, C * 8 * 128)], obuf, wsem).wait()


def _transpose(x):
    mesh = plsc.VectorSubcoreMesh(core_axis_name="c", subcore_axis_name="s")
    ft = pl.kernel(
        _tbody,
        out_type=jax.ShapeDtypeStruct((NPIX * C,), jnp.float32),
        mesh=mesh,
        compiler_params=pltpu.CompilerParams(
            needs_layout_passes=False, use_tc_tiling_on_sc=True),
        scratch_types=[
            pltpu.VMEM((2, C, 8, 128), jnp.float32),
            pltpu.VMEM((8 * 128 * C,), jnp.float32),
            pltpu.SemaphoreType.DMA,
            pltpu.SemaphoreType.DMA,
            pltpu.SemaphoreType.DMA,
        ],
    )
    return ft(x)


def kernel(x, g):
    xt = _transpose(x).reshape(NPIX, C)
    gx = jnp.reshape(g[..., 0], (NPIX,))
    gy = jnp.reshape(g[..., 1], (NPIX,))
    mesh = plsc.VectorSubcoreMesh(core_axis_name="c", subcore_axis_name="s")
    f = pl.kernel(
        _body,
        out_type=jax.ShapeDtypeStruct((N * C * HW,), jnp.float32),
        mesh=mesh,
        compiler_params=pltpu.CompilerParams(
            needs_layout_passes=False, use_tc_tiling_on_sc=False),
        scratch_types=[
            pltpu.VMEM((PPW,), jnp.float32),
            pltpu.VMEM((PPW,), jnp.float32),
            pltpu.VMEM((2, NSUB, 128), jnp.int32),
            pltpu.VMEM((2, NSUB, 128), jnp.int32),
            pltpu.VMEM((2, NSUB, 128), jnp.int32),
            pltpu.VMEM((2, NSUB, 128), jnp.int32),
            pltpu.VMEM((2, BP), jnp.float32),
            pltpu.VMEM((2, BP), jnp.float32),
            pltpu.VMEM((2, BP), jnp.float32),
            pltpu.VMEM((2, BP), jnp.float32),
            pltpu.VMEM((2, BP, C), jnp.float32),
            pltpu.VMEM((2, BP, C), jnp.float32),
            pltpu.VMEM((2, BP, C), jnp.float32),
            pltpu.VMEM((2, BP, C), jnp.float32),
            pltpu.VMEM((2, C * BP), jnp.float32),
            pltpu.SemaphoreType.DMA,
            pltpu.SemaphoreType.DMA,
            pltpu.SemaphoreType.DMA,
            pltpu.SemaphoreType.DMA,
        ],
    )
    y = f(xt, gx, gy)
    return y.reshape(N, C, H, W)


# speedup vs baseline: 5.0714x; 1.0085x over previous
import jax
import jax.numpy as jnp
from jax import lax
from jax.experimental import pallas as pl
from jax.experimental.pallas import tpu as pltpu, tpu_sc as plsc

N, C, H, W = 4, 32, 384, 384
HW = H * W
NPIX = N * HW
NC, NS, L = 2, 16, 16
NW = NC * NS
PPW = HW // NW
BP = 256
NBLK = PPW // BP
TBLK = N * NBLK
NSUB = BP // 128
NQ = BP // L


def _body(xt, gx, gy, out, gxv, gyv, i00, i10, i01, i11, w00, w10, w01, w11,
          v00, v10, v01, v11, ob, gsem0, gsem1, osem0, osem1):
    cid = lax.axis_index("c")
    sid = lax.axis_index("s")
    wid = sid * NC + cid
    iota = lax.iota(jnp.int32, L)
    gsems = (gsem0, gsem1)
    osems = (osem0, osem1)

    def fire(bp, tn):
        bat = tn // NBLK
        r = tn - bat * NBLK

        @pl.when(r == 0)
        def _():
            goff = pl.multiple_of(bat * HW + wid * PPW, PPW)
            pltpu.sync_copy(gx.at[pl.ds(goff, PPW)], gxv)
            pltpu.sync_copy(gy.at[pl.ds(goff, PPW)], gyv)

        rowbase = bat * HW

        def qstep(q, carry):
            sl = pl.ds(r * BP + q * L, L)
            ix = (gxv[sl] + 1.0) * (0.5 * (W - 1))
            iy = (gyv[sl] + 1.0) * (0.5 * (H - 1))
            tx = ix.astype(jnp.int32)
            ty = iy.astype(jnp.int32)
            ix0 = tx - jnp.where(ix < tx.astype(jnp.float32), 1, 0)
            iy0 = ty - jnp.where(iy < ty.astype(jnp.float32), 1, 0)
            wx1 = ix - ix0.astype(jnp.float32)
            wy1 = iy - iy0.astype(jnp.float32)
            wx0 = 1.0 - wx1
            wy0 = 1.0 - wy1
            vx0 = jnp.where((ix0 >= 0) & (ix0 <= W - 1), 1.0, 0.0)
            vx1 = jnp.where((ix0 >= -1) & (ix0 <= W - 2), 1.0, 0.0)
            vy0 = jnp.where((iy0 >= 0) & (iy0 <= H - 1), 1.0, 0.0)
            vy1 = jnp.where((iy0 >= -1) & (iy0 <= H - 2), 1.0, 0.0)
            cx0 = jnp.clip(ix0, 0, W - 1)
            cx1 = jnp.clip(ix0 + 1, 0, W - 1)
            r0 = rowbase + jnp.clip(iy0, 0, H - 1) * W
            r1 = rowbase + jnp.clip(iy0 + 1, 0, H - 1) * W
            sub = q // (128 // L)
            cc = (q - sub * (128 // L)) * L
            csl = pl.ds(cc, L)
            i00[bp, sub, csl] = r0 + cx0
            i10[bp, sub, csl] = r0 + cx1
            i01[bp, sub, csl] = r1 + cx0
            i11[bp, sub, csl] = r1 + cx1
            qsl = pl.ds(q * L, L)
            w00[bp, qsl] = wx0 * wy0 * vx0 * vy0
            w10[bp, qsl] = wx1 * wy0 * vx1 * vy0
            w01[bp, qsl] = wx0 * wy1 * vx0 * vy1
            w11[bp, qsl] = wx1 * wy1 * vx1 * vy1
            return carry

        lax.fori_loop(0, NQ, qstep, 0)
        for i in range(NSUB):
            dsl = pl.ds(i * 128, 128)
            for iref, vref in ((i00, v00), (i10, v10), (i01, v01), (i11, v11)):
                pltpu.async_copy(xt.at[iref.at[bp, i]], vref.at[bp, dsl],
                                 gsems[bp])

    def drain_gathers(bp):
        for vref in (v00, v10, v01, v11):
            pltpu.make_async_copy(xt.at[pl.ds(0, BP)], vref.at[bp],
                                  gsems[bp]).wait()

    def combine(bp, t):
        parv = jnp.full((L,), bp, jnp.int32)

        def jstep(j, carry):
            rows = j * L + iota
            jsl = pl.ds(j * L, L)
            w00v = w00[bp, jsl]
            w10v = w10[bp, jsl]
            w01v = w01[bp, jsl]
            w11v = w11[bp, jsl]
            for c0 in range(C):
                colv = (c0 + iota) & (C - 1)
                a = plsc.load_gather(v00, [parv, rows, colv]) * w00v
                a = a + plsc.load_gather(v10, [parv, rows, colv]) * w10v
                a = a + plsc.load_gather(v01, [parv, rows, colv]) * w01v
                a = a + plsc.load_gather(v11, [parv, rows, colv]) * w11v
                plsc.store_scatter(ob, [parv, colv * BP + rows], a)
            return carry

        lax.fori_loop(0, NQ, jstep, 0)

    def fire_outs(bp, t):
        bat = t // NBLK
        q0 = (t - bat * NBLK) * BP + wid * PPW
        for c in range(C):
            off = pl.multiple_of((bat * C + c) * HW + q0, BP)
            pltpu.async_copy(ob.at[bp, pl.ds(c * BP, BP)],
                             out.at[pl.ds(off, BP)], osems[bp])

    def drain_outs(bp):
        pltpu.make_async_copy(out.at[pl.ds(0, C * BP)], ob.at[bp],
                              osems[bp]).wait()

    fire(0, 0)

    def block(t, carry):
        for par in range(2):
            @pl.when(lax.rem(t, 2) == par)
            def _():
                @pl.when(t >= 2)
                def _():
                    drain_outs(par)

                @pl.when(t + 1 < TBLK)
                def _():
                    fire(1 - par, t + 1)

                drain_gathers(par)
                combine(par, t)
                fire_outs(par, t)
        return carry

    lax.fori_loop(0, TBLK, block, 0)
    drain_outs(TBLK % 2)
    drain_outs(1 - TBLK % 2)


TB = 18
RING = 15
CHUNK = 128 * C


def _tbody(x, out, ibuf, obuf, rsem0, rsem1, wsem):
    cid = lax.axis_index("c")
    sid = lax.axis_index("s")
    wid = sid * NC + cid
    iota = lax.iota(jnp.int32, L)
    rsems = (rsem0, rsem1)
    WT = W // 128
    PERN = (H // 8) * WT

    def coords(b):
        blk = wid * TB + b
        n = blk // PERN
        rem = blk - n * PERN
        hs = rem // WT
        wt = rem - hs * WT
        return n, hs * 8, wt * 128

    def fire_reads(bp, b):
        n, h0, w0 = coords(b)
        for c in range(C):
            pltpu.async_copy(x.at[n, c, pl.ds(h0, 8), pl.ds(w0, 128)],
                             ibuf.at[bp, c], rsems[bp])

    def drain_reads(bp):
        pltpu.make_async_copy(x.at[0, pl.ds(0, C), pl.ds(0, 8), pl.ds(0, 128)],
                              ibuf.at[bp], rsems[bp]).wait()

    def drain_chunk():
        pltpu.make_async_copy(out.at[pl.ds(0, CHUNK)],
                              obuf.at[pl.ds(0, CHUNK)], wsem).wait()

    fire_reads(0, 0)

    def block(b, carry):
        for p in range(2):
            @pl.when(lax.rem(b, 2) == p)
            def _():
                @pl.when(b + 1 < TB)
                def _():
                    fire_reads(1 - p, b + 1)

                drain_reads(p)

        parv = jnp.full((L,), lax.rem(b, 2), jnp.int32)
        n, h0, w0 = coords(b)

        def hstep(h, carry2):
            g = b * 8 + h
            slot = lax.rem(g, RING)

            @pl.when(g >= RING)
            def _():
                drain_chunk()

            hv = jnp.full((L,), h, jnp.int32)
            sbase = pl.multiple_of(slot * CHUNK, CHUNK)

            def qstep(q, carry3):
                wv = q * L + iota
                obase = sbase + wv * C
                for c0 in range(C):
                    cv = (c0 + iota) & (C - 1)
                    a = plsc.load_gather(ibuf, [parv, cv, hv, wv])
                    plsc.store_scatter(obuf, [obase + cv], a)
                return carry3

            lax.fori_loop(0, 8, qstep, 0)
            off = pl.multiple_of((n * HW + (h0 + h) * W + w0) * C, CHUNK)
            pltpu.async_copy(obuf.at[pl.ds(sbase, CHUNK)],
                             out.at[pl.ds(off, CHUNK)], wsem)
            return carry2

        lax.fori_loop(0, 8, hstep, 0)
        return carry

    lax.fori_loop(0, TB, block, 0)
    for _ in range(RING):
        drain_chunk()


def _transpose(x):
    mesh = plsc.VectorSubcoreMesh(core_axis_name="c", subcore_axis_name="s")
    ft = pl.kernel(
        _tbody,
        out_type=jax.ShapeDtypeStruct((NPIX * C,), jnp.float32),
        mesh=mesh,
        compiler_params=pltpu.CompilerParams(
            needs_layout_passes=False, use_tc_tiling_on_sc=True),
        scratch_types=[
            pltpu.VMEM((2, C, 8, 128), jnp.float32),
            pltpu.VMEM((RING * CHUNK,), jnp.float32),
            pltpu.SemaphoreType.DMA,
            pltpu.SemaphoreType.DMA,
            pltpu.SemaphoreType.DMA,
        ],
    )
    return ft(x)


def kernel(x, g):
    xt = _transpose(x).reshape(NPIX, C)
    gx = jnp.reshape(g[..., 0], (NPIX,))
    gy = jnp.reshape(g[..., 1], (NPIX,))
    mesh = plsc.VectorSubcoreMesh(core_axis_name="c", subcore_axis_name="s")
    f = pl.kernel(
        _body,
        out_type=jax.ShapeDtypeStruct((N * C * HW,), jnp.float32),
        mesh=mesh,
        compiler_params=pltpu.CompilerParams(
            needs_layout_passes=False, use_tc_tiling_on_sc=False),
        scratch_types=[
            pltpu.VMEM((PPW,), jnp.float32),
            pltpu.VMEM((PPW,), jnp.float32),
            pltpu.VMEM((2, NSUB, 128), jnp.int32),
            pltpu.VMEM((2, NSUB, 128), jnp.int32),
            pltpu.VMEM((2, NSUB, 128), jnp.int32),
            pltpu.VMEM((2, NSUB, 128), jnp.int32),
            pltpu.VMEM((2, BP), jnp.float32),
            pltpu.VMEM((2, BP), jnp.float32),
            pltpu.VMEM((2, BP), jnp.float32),
            pltpu.VMEM((2, BP), jnp.float32),
            pltpu.VMEM((2, BP, C), jnp.float32),
            pltpu.VMEM((2, BP, C), jnp.float32),
            pltpu.VMEM((2, BP, C), jnp.float32),
            pltpu.VMEM((2, BP, C), jnp.float32),
            pltpu.VMEM((2, C * BP), jnp.float32),
            pltpu.SemaphoreType.DMA,
            pltpu.SemaphoreType.DMA,
            pltpu.SemaphoreType.DMA,
            pltpu.SemaphoreType.DMA,
        ],
    )
    y = f(xt, gx, gy)
    return y.reshape(N, C, H, W)
